# baseline (speedup 1.0000x reference)
import functools

import numpy as np
import jax
import jax.numpy as jnp
from jax import lax
from jax.experimental import pallas as pl
from jax.experimental.pallas import tpu as pltpu
from jax.experimental.pallas import tpu_sc as plsc

B = 4096
S = 26
D = 64
V = 1000000
NT = S + 1
T0, T1 = 512, 256
H0, H1 = 512, 256
NW = 32
ROWS = B * S
CHUNK = 128
CPW = ROWS // (NW * CHUNK)
PW = ROWS // NW

_sc_mesh = plsc.VectorSubcoreMesh(core_axis_name="c", subcore_axis_name="s")


@functools.partial(
    pl.kernel,
    mesh=_sc_mesh,
    out_type=jax.ShapeDtypeStruct((ROWS, D), jnp.float32),
    scratch_types=[
        pltpu.VMEM((CPW, CHUNK), jnp.int32),
        pltpu.VMEM((CHUNK, D), jnp.float32),
        pltpu.SemaphoreType.DMA,
    ],
    compiler_params=pltpu.CompilerParams(use_tc_tiling_on_sc=False),
)
def _sc_gather(table_hbm, idx_hbm, out_hbm, idx_v, rows_v, sem):
    wid = lax.axis_index("s") * 2 + lax.axis_index("c")
    pltpu.sync_copy(idx_hbm.at[wid], idx_v)
    base = wid * PW

    def body(j, carry):
        pltpu.async_copy(table_hbm.at[idx_v.at[j]], rows_v, sem).wait()
        pltpu.sync_copy(rows_v, out_hbm.at[pl.ds(base + j * CHUNK, CHUNK)])
        return carry

    lax.fori_loop(0, CPW, body, 0)


BLK = 1024


def _bot_body(dxt, w0, b0, w1, b1, w2, b2, out):
    x = jnp.maximum(w0[...] @ dxt[...] + b0[...], 0.0)
    x = jnp.maximum(w1[...] @ x + b1[...], 0.0)
    out[...] = jnp.maximum(w2[...] @ x + b2[...], 0.0)


def _bot_call(dxt, w0, b0, w1, b1, w2, b2):
    full = lambda shape: pl.BlockSpec(shape, lambda i: (0, 0))
    return pl.pallas_call(
        _bot_body,
        grid=(B // BLK,),
        in_specs=[
            pl.BlockSpec((13, BLK), lambda i: (0, i)),
            full((H0, 13)), full((H0, 1)),
            full((H1, H0)), full((H1, 1)),
            full((D, H1)), full((D, 1)),
        ],
        out_specs=pl.BlockSpec((D, BLK), lambda i: (0, i)),
        out_shape=jax.ShapeDtypeStruct((D, B), jnp.float32),
    )(dxt, w0, b0, w1, b1, w2, b2)


def _top_body(xt, embt, w0a, wpt, tb0, tw1, tb1, tw2, tb2, out):
    x = xt[...]
    E = embt[...]
    acc = w0a[...] @ x + tb0[...]
    for m in range(S):
        tm = x if m == 0 else E[m - 1]
        z = jnp.sum(E[m:] * tm[None, :, :], axis=1)
        acc = acc + wpt[m][:, m:] @ z
    y = jnp.maximum(acc, 0.0)
    y = jnp.maximum(tw1[...] @ y + tb1[...], 0.0)
    out[...] = jax.nn.sigmoid(tw2[...] @ y + tb2[...])


def _top_call(xt, embt, w0a, wpt, tb0, tw1, tb1, tw2, tb2):
    full = lambda shape: pl.BlockSpec(shape, lambda i: tuple(0 for _ in shape))
    return pl.pallas_call(
        _top_body,
        grid=(B // BLK,),
        in_specs=[
            pl.BlockSpec((D, BLK), lambda i: (0, i)),
            pl.BlockSpec((S, D, BLK), lambda i: (0, 0, i)),
            full((T0, D)),
            full((S, T0, S)),
            full((T0, 1)),
            full((T1, T0)), full((T1, 1)),
            full((1, T1)), full((1, 1)),
        ],
        out_specs=pl.BlockSpec((1, BLK), lambda i: (0, i)),
        out_shape=jax.ShapeDtypeStruct((1, B), jnp.float32),
    )(xt, embt, w0a, wpt, tb0, tw1, tb1, tw2, tb2)


_NI, _NJ = np.tril_indices(NT, -1)


def kernel(dense_x, emb_table, bot_W0, bot_b0, bot_W1, bot_b1, bot_W2, bot_b2,
           top_W0, top_b0, top_W1, top_b1, top_W2, top_b2, sparse_idx):
    si = sparse_idx.astype(jnp.int32).T
    idx2 = si.reshape(NW, CPW, CHUNK)
    emb_flat = _sc_gather(emb_table, idx2)
    embt = jnp.swapaxes(emb_flat.reshape(S, B, D), 1, 2)

    xt = _bot_call(dense_x.T, bot_W0, bot_b0.reshape(-1, 1),
                   bot_W1, bot_b1.reshape(-1, 1),
                   bot_W2, bot_b2.reshape(-1, 1))

    wpt = jnp.zeros((S, T0, S), jnp.float32).at[_NJ, :, _NI - 1].set(top_W0[:, D:].T)
    w0a = top_W0[:, :D]

    pt = _top_call(xt, embt, w0a, wpt, top_b0.reshape(-1, 1),
                   top_W1, top_b1.reshape(-1, 1),
                   top_W2, top_b2.reshape(-1, 1))
    return pt.reshape(B, 1)


# speedup vs baseline: 1.0087x; 1.0087x over previous
import functools

import numpy as np
import jax
import jax.numpy as jnp
from jax import lax
from jax.experimental import pallas as pl
from jax.experimental.pallas import tpu as pltpu
from jax.experimental.pallas import tpu_sc as plsc

B = 4096
S = 26
D = 64
V = 1000000
NT = S + 1
T0, T1 = 512, 256
H0, H1 = 512, 256
NW = 32
ROWS = B * S
CHUNK = 128
CPW = ROWS // (NW * CHUNK)
PW = ROWS // NW

_sc_mesh = plsc.VectorSubcoreMesh(core_axis_name="c", subcore_axis_name="s")


@functools.partial(
    pl.kernel,
    mesh=_sc_mesh,
    out_type=jax.ShapeDtypeStruct((ROWS, D), jnp.float32),
    scratch_types=[
        pltpu.VMEM((CPW, CHUNK), jnp.int32),
        pltpu.VMEM((2, CHUNK, D), jnp.float32),
        pltpu.SemaphoreType.DMA,
        pltpu.SemaphoreType.DMA,
    ],
    compiler_params=pltpu.CompilerParams(use_tc_tiling_on_sc=False),
)
def _sc_gather(table_hbm, idx_hbm, out_hbm, idx_v, rows_v, sem0, sem1):
    wid = lax.axis_index("s") * 2 + lax.axis_index("c")
    pltpu.sync_copy(idx_hbm.at[wid], idx_v)
    base = wid * PW

    pltpu.async_copy(table_hbm.at[idx_v.at[0]], rows_v.at[0], sem0)

    def body(j2, carry):
        je = j2 * 2
        jo = je + 1
        pltpu.make_async_copy(table_hbm.at[idx_v.at[je]], rows_v.at[0],
                              sem0).wait()
        pltpu.async_copy(table_hbm.at[idx_v.at[jo]], rows_v.at[1], sem1)
        pltpu.sync_copy(rows_v.at[0],
                        out_hbm.at[pl.ds(base + je * CHUNK, CHUNK)])
        pltpu.make_async_copy(table_hbm.at[idx_v.at[jo]], rows_v.at[1],
                              sem1).wait()

        @pl.when(j2 * 2 + 2 < CPW)
        def _():
            pltpu.async_copy(table_hbm.at[idx_v.at[je + 2]], rows_v.at[0],
                             sem0)

        pltpu.sync_copy(rows_v.at[1],
                        out_hbm.at[pl.ds(base + jo * CHUNK, CHUNK)])
        return carry

    lax.fori_loop(0, CPW // 2, body, 0)


BLK = 1024


def _bot_body(dxt, w0, b0, w1, b1, w2, b2, out):
    x = jnp.maximum(w0[...] @ dxt[...] + b0[...], 0.0)
    x = jnp.maximum(w1[...] @ x + b1[...], 0.0)
    out[...] = jnp.maximum(w2[...] @ x + b2[...], 0.0)


def _bot_call(dxt, w0, b0, w1, b1, w2, b2):
    full = lambda shape: pl.BlockSpec(shape, lambda i: (0, 0))
    return pl.pallas_call(
        _bot_body,
        grid=(B // BLK,),
        in_specs=[
            pl.BlockSpec((13, BLK), lambda i: (0, i)),
            full((H0, 13)), full((H0, 1)),
            full((H1, H0)), full((H1, 1)),
            full((D, H1)), full((D, 1)),
        ],
        out_specs=pl.BlockSpec((D, BLK), lambda i: (0, i)),
        out_shape=jax.ShapeDtypeStruct((D, B), jnp.float32),
    )(dxt, w0, b0, w1, b1, w2, b2)


def _top_body(xt, embt, w0a, wpt, tb0, tw1, tb1, tw2, tb2, out):
    x = xt[...]
    E = embt[...]
    acc = w0a[...] @ x + tb0[...]
    for m in range(S):
        tm = x if m == 0 else E[m - 1]
        z = jnp.sum(E[m:] * tm[None, :, :], axis=1)
        acc = acc + wpt[m][:, m:] @ z
    y = jnp.maximum(acc, 0.0)
    y = jnp.maximum(tw1[...] @ y + tb1[...], 0.0)
    out[...] = jax.nn.sigmoid(tw2[...] @ y + tb2[...])


def _top_call(xt, embt, w0a, wpt, tb0, tw1, tb1, tw2, tb2):
    full = lambda shape: pl.BlockSpec(shape, lambda i: tuple(0 for _ in shape))
    return pl.pallas_call(
        _top_body,
        grid=(B // BLK,),
        in_specs=[
            pl.BlockSpec((D, BLK), lambda i: (0, i)),
            pl.BlockSpec((S, D, BLK), lambda i: (0, 0, i)),
            full((T0, D)),
            full((S, T0, S)),
            full((T0, 1)),
            full((T1, T0)), full((T1, 1)),
            full((1, T1)), full((1, 1)),
        ],
        out_specs=pl.BlockSpec((1, BLK), lambda i: (0, i)),
        out_shape=jax.ShapeDtypeStruct((1, B), jnp.float32),
    )(xt, embt, w0a, wpt, tb0, tw1, tb1, tw2, tb2)


_NI, _NJ = np.tril_indices(NT, -1)


def kernel(dense_x, emb_table, bot_W0, bot_b0, bot_W1, bot_b1, bot_W2, bot_b2,
           top_W0, top_b0, top_W1, top_b1, top_W2, top_b2, sparse_idx):
    si = sparse_idx.astype(jnp.int32).T
    idx2 = si.reshape(NW, CPW, CHUNK)
    emb_flat = _sc_gather(emb_table, idx2)
    embt = jnp.swapaxes(emb_flat.reshape(S, B, D), 1, 2)

    xt = _bot_call(dense_x.T, bot_W0, bot_b0.reshape(-1, 1),
                   bot_W1, bot_b1.reshape(-1, 1),
                   bot_W2, bot_b2.reshape(-1, 1))

    wpt = jnp.zeros((S, T0, S), jnp.float32).at[_NJ, :, _NI - 1].set(top_W0[:, D:].T)
    w0a = top_W0[:, :D]

    pt = _top_call(xt, embt, w0a, wpt, top_b0.reshape(-1, 1),
                   top_W1, top_b1.reshape(-1, 1),
                   top_W2, top_b2.reshape(-1, 1))
    return pt.reshape(B, 1)
